# Initial kernel scaffold; baseline (speedup 1.0000x reference)
import functools

import jax
import jax.numpy as jnp
from jax import lax
from jax.experimental import pallas as pl
from jax.experimental.pallas import tpu as pltpu
from jax.experimental.pallas import tpu_sc as plsc

NODES = 50000
NUSERS = 25000
D = 64
DH = 32
E = 800000
CH = 128
CHUNKS = 6272
EPAD = CHUNKS * CH
TILES = 16
CPT = CHUNKS // TILES
K = 8
BLOCKS = CPT // K
ROWS_PT = NODES // TILES
WB = 625
NWB = ROWS_PT // WB
ACC_ROWS = NODES + 8

_mesh = plsc.VectorSubcoreMesh(core_axis_name="c", subcore_axis_name="s")


@functools.partial(
    pl.kernel,
    mesh=_mesh,
    out_type=(
        jax.ShapeDtypeStruct((2 * NODES, DH), jnp.float32),
        jax.ShapeDtypeStruct((2 * NODES, DH), jnp.float32),
    ),
    scratch_types=[
        pltpu.VMEM((K, CH), jnp.int32),
        pltpu.VMEM((K, CH), jnp.int32),
        pltpu.VMEM((K, CH, DH), jnp.float32),
        pltpu.VMEM((WB, DH), jnp.float32),
        pltpu.VMEM((WB, DH), jnp.float32),
        pltpu.VMEM_SHARED((ACC_ROWS, DH), jnp.float32),
        pltpu.SemaphoreType.DMA,
    ],
)
def _propagate(tab0, rows_hbm, cols_hbm, zeros_hbm, x1_out, x2_out,
               rows_v, cols_v, gbuf, zbuf, wbuf, acc, sem):
    c = lax.axis_index("c")
    s = lax.axis_index("s")
    coff = c * NODES

    pltpu.sync_copy(zeros_hbm, zbuf)

    def zero_acc():
        for k in range(NWB):
            pltpu.sync_copy(zbuf, acc.at[pl.ds(s * ROWS_PT + k * WB, WB)])

    def layer(src_tab, dst_tab):
        zero_acc()
        plsc.subcore_barrier()

        def block(b, carry):
            chunk0 = s * CPT + b * K
            pltpu.sync_copy(rows_hbm.at[pl.ds(chunk0, K)], rows_v)
            pltpu.sync_copy(cols_hbm.at[pl.ds(chunk0, K)], cols_v)
            for j in range(K):
                for t in range(CH // 16):
                    sl = pl.ds(t * 16, 16)
                    cols_v[j, sl] = cols_v[j, sl] + coff
            cps = [
                pltpu.async_copy(src_tab.at[cols_v.at[j]], gbuf.at[j], sem)
                for j in range(K)
            ]
            for cp in cps:
                cp.wait()
            for j in range(K):
                pltpu.sync_copy(gbuf.at[j], acc.at[rows_v.at[j]], add=True)
            return carry

        lax.fori_loop(0, BLOCKS, block, 0)
        plsc.subcore_barrier()
        for k in range(NWB):
            r0 = s * ROWS_PT + k * WB
            pltpu.sync_copy(acc.at[pl.ds(r0, WB)], wbuf)
            pltpu.sync_copy(wbuf, dst_tab.at[pl.ds(coff + r0, WB)])
        plsc.subcore_barrier()

    layer(tab0, x1_out)
    layer(x1_out, x2_out)


def _mean_body(x0_ref, x1_ref, x2_ref, o_ref):
    o_ref[...] = (x0_ref[...] + x1_ref[...] + x2_ref[...]) * (1.0 / 3.0)


_MEAN_BLOCK = 3125


def _mean3(x0, x1, x2):
    n = x0.shape[0]
    grid = n // _MEAN_BLOCK
    spec = pl.BlockSpec((_MEAN_BLOCK, 128), lambda i: (i, 0))
    return pl.pallas_call(
        _mean_body,
        grid=(grid,),
        in_specs=[spec, spec, spec],
        out_specs=spec,
        out_shape=jax.ShapeDtypeStruct((n, 128), jnp.float32),
    )(x0, x1, x2)


def kernel(edge_index, emb_weight):
    row = edge_index[0]
    col = edge_index[1]
    pad = EPAD - E
    rows = jnp.concatenate(
        [row, jnp.full((pad,), NODES, jnp.int32)]).reshape(CHUNKS, CH)
    cols = jnp.concatenate(
        [col, jnp.zeros((pad,), jnp.int32)]).reshape(CHUNKS, CH)
    tab0 = jnp.concatenate([emb_weight[:, :DH], emb_weight[:, DH:]], axis=0)
    zeros = jnp.zeros((WB, DH), jnp.float32)

    x1_tab, x2_tab = _propagate(tab0, rows, cols, zeros)

    zt = _mean3(tab0.reshape(2 * NUSERS, 128),
                x1_tab.reshape(2 * NUSERS, 128),
                x2_tab.reshape(2 * NUSERS, 128)).reshape(2 * NODES, DH)
    z = jnp.concatenate([zt[:NODES], zt[NODES:]], axis=1)
    return z[:NUSERS], z[NUSERS:NODES]


# speedup vs baseline: 7.8716x; 7.8716x over previous
import functools

import jax
import jax.numpy as jnp
from jax import lax
from jax.experimental import pallas as pl
from jax.experimental.pallas import tpu as pltpu
from jax.experimental.pallas import tpu_sc as plsc

NODES = 50000
NP = 50048
NUSERS = 25000
D = 64
DH = 32
E = 800000
CH = 128
CHUNKS = 6272
EPAD = CHUNKS * CH
TILES = 16
CPT = CHUNKS // TILES
K = 4
BLOCKS = CPT // K
ROWS_PT = NP // TILES
WB = 136
NWB = ROWS_PT // WB

_mesh = plsc.VectorSubcoreMesh(core_axis_name="c", subcore_axis_name="s")


@functools.partial(
    pl.kernel,
    mesh=_mesh,
    compiler_params=pltpu.CompilerParams(use_tc_tiling_on_sc=False),
    out_type=(
        jax.ShapeDtypeStruct((2 * NP, DH), jnp.float32),
        jax.ShapeDtypeStruct((2 * NP, DH), jnp.float32),
    ),
    scratch_types=[
        pltpu.VMEM((K, CH), jnp.int32),
        pltpu.VMEM((K, CH), jnp.int32),
        pltpu.VMEM((K, CH, DH), jnp.float32),
        pltpu.VMEM((WB, DH), jnp.float32),
        pltpu.VMEM((WB, DH), jnp.float32),
        pltpu.VMEM_SHARED((NP, DH), jnp.float32),
        pltpu.SemaphoreType.DMA,
    ],
)
def _propagate(tab0, rows_hbm, cols_hbm, zeros_hbm, x1_out, x2_out,
               rows_v, cols_v, gbuf, zbuf, wbuf, acc, sem):
    c = lax.axis_index("c")
    s = lax.axis_index("s")
    coff = c * NP

    pltpu.sync_copy(zeros_hbm, zbuf)

    def zero_acc():
        for k in range(NWB):
            pltpu.sync_copy(zbuf, acc.at[pl.ds(s * ROWS_PT + k * WB, WB)])

    def layer(src_tab, dst_tab):
        zero_acc()
        plsc.subcore_barrier()

        def block(b, carry):
            chunk0 = s * CPT + b * K
            pltpu.sync_copy(rows_hbm.at[pl.ds(chunk0, K)], rows_v)
            pltpu.sync_copy(cols_hbm.at[pl.ds(chunk0, K)], cols_v)
            for j in range(K):
                for t in range(CH // 16):
                    sl = pl.ds(t * 16, 16)
                    cols_v[j, sl] = cols_v[j, sl] + coff
            cps = [
                pltpu.async_copy(src_tab.at[cols_v.at[j]], gbuf.at[j], sem)
                for j in range(K)
            ]
            for cp in cps:
                cp.wait()
            for j in range(K):
                pltpu.sync_copy(gbuf.at[j], acc.at[rows_v.at[j]], add=True)
            return carry

        lax.fori_loop(0, BLOCKS, block, 0)
        plsc.subcore_barrier()
        for k in range(NWB):
            r0 = s * ROWS_PT + k * WB
            pltpu.sync_copy(acc.at[pl.ds(r0, WB)], wbuf)
            pltpu.sync_copy(wbuf, dst_tab.at[pl.ds(coff + r0, WB)])
        plsc.subcore_barrier()

    layer(tab0, x1_out)
    layer(x1_out, x2_out)


def _mean_body(x0_ref, x1_ref, x2_ref, o_ref):
    o_ref[...] = (x0_ref[...] + x1_ref[...] + x2_ref[...]) * (1.0 / 3.0)


_MEAN_BLOCK = 3128


def _mean3(x0, x1, x2):
    n = x0.shape[0]
    grid = n // _MEAN_BLOCK
    spec = pl.BlockSpec((_MEAN_BLOCK, 128), lambda i: (i, 0))
    return pl.pallas_call(
        _mean_body,
        grid=(grid,),
        in_specs=[spec, spec, spec],
        out_specs=spec,
        out_shape=jax.ShapeDtypeStruct((n, 128), jnp.float32),
    )(x0, x1, x2)


def kernel(edge_index, emb_weight):
    row = edge_index[0]
    col = edge_index[1]
    pad = EPAD - E
    rows = jnp.concatenate(
        [row, jnp.full((pad,), NODES, jnp.int32)]).reshape(CHUNKS, CH)
    cols = jnp.concatenate(
        [col, jnp.zeros((pad,), jnp.int32)]).reshape(CHUNKS, CH)
    embp = jnp.pad(emb_weight, ((0, NP - NODES), (0, 0)))
    tab0 = jnp.concatenate([embp[:, :DH], embp[:, DH:]], axis=0)
    zeros = jnp.zeros((WB, DH), jnp.float32)

    x1_tab, x2_tab = _propagate(tab0, rows, cols, zeros)

    zt = _mean3(tab0.reshape(2 * NP * DH // 128, 128),
                x1_tab.reshape(2 * NP * DH // 128, 128),
                x2_tab.reshape(2 * NP * DH // 128, 128)).reshape(2 * NP, DH)
    z = jnp.concatenate([zt[:NODES], zt[NP:NP + NODES]], axis=1)
    return z[:NUSERS], z[NUSERS:NODES]
